# baseline (speedup 1.0000x reference)
import functools

import jax
import jax.numpy as jnp
from jax import lax
from jax.experimental import pallas as pl
from jax.experimental.pallas import tpu as pltpu
from jax.experimental.pallas import tpu_sc as plsc

LANES = 16


@functools.lru_cache(maxsize=None)
def _build(B, S, V, D):
    info = plsc.get_sparse_core_info()
    NC, NS = info.num_cores, info.num_subcores
    NW = NC * NS
    s_per_w = S // NW
    b_per_w = B * s_per_w
    assert S % NW == 0 and s_per_w % 8 == 0 and s_per_w <= 128
    assert D % LANES == 0

    mesh = plsc.VectorSubcoreMesh(core_axis_name="c", subcore_axis_name="s")

    @functools.partial(
        pl.kernel,
        mesh=mesh,
        out_type=jax.ShapeDtypeStruct((B, S, D), jnp.float32),
        scratch_types=[
            pltpu.VMEM((b_per_w,), jnp.int32),
            pltpu.VMEM((b_per_w, D), jnp.float32),
            pltpu.VMEM((s_per_w, D), jnp.float32),
            [pltpu.SemaphoreType.DMA] * B,
            pltpu.SemaphoreType.DMA,
            pltpu.SemaphoreType.DMA,
            pltpu.SemaphoreType.DMA,
        ],
    )
    def k(idx_hbm, wte_hbm, wpe_hbm, out_hbm, idx_v, rows_v, wpe_v, sem_g,
          sem_o, sem_i, sem_w):
        wid = lax.axis_index("s") * NC + lax.axis_index("c")
        s0 = wid * s_per_w
        idx_cp = []
        for b in range(B):
            idx_cp.append(
                pltpu.async_copy(
                    idx_hbm.at[b, pl.ds(s0, s_per_w)],
                    idx_v.at[pl.ds(b * s_per_w, s_per_w)],
                    sem_i,
                )
            )
        wpe_cp = pltpu.async_copy(
            wpe_hbm.at[pl.ds(s0, s_per_w)], wpe_v, sem_w
        )
        for c in idx_cp:
            c.wait()
        g_cp = []
        for b in range(B):
            g_cp.append(
                pltpu.async_copy(
                    wte_hbm.at[idx_v.at[pl.ds(b * s_per_w, s_per_w)]],
                    rows_v.at[pl.ds(b * s_per_w, s_per_w)],
                    sem_g[b],
                )
            )
        wpe_cp.wait()
        nvec = D // LANES
        UNROLL = 4
        o_cp = []
        for b in range(B):
            g_cp[b].wait()

            def add_rows(i, c, b=b):
                for u in range(UNROLL):
                    r = i * UNROLL + u
                    for j in range(nvec):
                        sl = pl.ds(j * LANES, LANES)
                        rows_v[b * s_per_w + r, sl] = (
                            rows_v[b * s_per_w + r, sl] + wpe_v[r, sl]
                        )
                return c

            lax.fori_loop(0, s_per_w // UNROLL, add_rows, 0)
            o_cp.append(
                pltpu.async_copy(
                    rows_v.at[pl.ds(b * s_per_w, s_per_w)],
                    out_hbm.at[b, pl.ds(s0, s_per_w)],
                    sem_o,
                )
            )
        for c in o_cp:
            c.wait()

    return k


def kernel(idx, wte, wpe):
    B, S = idx.shape
    V, D = wte.shape
    return _build(B, S, V, D)(idx.astype(jnp.int32), wte, wpe)


# speedup vs baseline: 1.0590x; 1.0590x over previous
import functools

import jax
import jax.numpy as jnp
from jax import lax
from jax.experimental import pallas as pl
from jax.experimental.pallas import tpu as pltpu
from jax.experimental.pallas import tpu_sc as plsc

LANES = 16


@functools.lru_cache(maxsize=None)
def _build(B, S, V, D):
    info = plsc.get_sparse_core_info()
    NC, NS = info.num_cores, info.num_subcores
    NW = NC * NS
    s_per_w = S // NW
    b_per_w = B * s_per_w
    assert S % NW == 0 and s_per_w % 8 == 0 and s_per_w <= 128
    assert D % LANES == 0

    mesh = plsc.VectorSubcoreMesh(core_axis_name="c", subcore_axis_name="s")

    @functools.partial(
        pl.kernel,
        mesh=mesh,
        out_type=jax.ShapeDtypeStruct((B, S, D), jnp.float32),
        scratch_types=[
            pltpu.VMEM((b_per_w,), jnp.int32),
            pltpu.VMEM((b_per_w, D), jnp.float32),
            pltpu.VMEM((s_per_w, D), jnp.float32),
            [pltpu.SemaphoreType.DMA] * B,
            pltpu.SemaphoreType.DMA,
            pltpu.SemaphoreType.DMA,
            pltpu.SemaphoreType.DMA,
        ],
    )
    def k(idx_hbm, wte_hbm, wpe_hbm, out_hbm, idx_v, rows_v, wpe_v, sem_g,
          sem_o, sem_i, sem_w):
        wid = lax.axis_index("s") * NC + lax.axis_index("c")
        s0 = wid * s_per_w
        idx_cp = []
        for b in range(B):
            idx_cp.append(
                pltpu.async_copy(
                    idx_hbm.at[b, pl.ds(s0, s_per_w)],
                    idx_v.at[pl.ds(b * s_per_w, s_per_w)],
                    sem_i,
                )
            )
        w0_cp = pltpu.async_copy(
            wpe_hbm.at[pl.ds(s0, s_per_w)], rows_v.at[pl.ds(0, s_per_w)],
            sem_w,
        )
        wv_cp = pltpu.async_copy(
            wpe_hbm.at[pl.ds(s0, s_per_w)], wpe_v, sem_o
        )
        for c in idx_cp:
            c.wait()
        w0_cp.wait()
        g_cp = [
            pltpu.async_copy(
                wte_hbm.at[idx_v.at[pl.ds(0, s_per_w)]],
                rows_v.at[pl.ds(0, s_per_w)],
                sem_g[0],
                add=True,
            )
        ]
        wv_cp.wait()

        nvec = D // LANES
        for b in range(1, B):
            def dup_row(i, c, b=b):
                for j in range(nvec):
                    rows_v[b * s_per_w + i, pl.ds(j * LANES, LANES)] = (
                        wpe_v[i, pl.ds(j * LANES, LANES)]
                    )
                return c
            lax.fori_loop(0, s_per_w, dup_row, 0)
            g_cp.append(
                pltpu.async_copy(
                    wte_hbm.at[idx_v.at[pl.ds(b * s_per_w, s_per_w)]],
                    rows_v.at[pl.ds(b * s_per_w, s_per_w)],
                    sem_g[b],
                    add=True,
                )
            )
        o_cp = []
        for b in range(B):
            g_cp[b].wait()
            o_cp.append(
                pltpu.async_copy(
                    rows_v.at[pl.ds(b * s_per_w, s_per_w)],
                    out_hbm.at[b, pl.ds(s0, s_per_w)],
                    sem_o,
                )
            )
        for c in o_cp:
            c.wait()

    return k


def kernel(idx, wte, wpe):
    B, S = idx.shape
    V, D = wte.shape
    return _build(B, S, V, D)(idx.astype(jnp.int32), wte, wpe)
